# baseline (speedup 1.0000x reference)
import functools

import jax
import jax.numpy as jnp
from jax import lax
from jax.experimental import pallas as pl
from jax.experimental.pallas import tpu as pltpu
from jax.experimental.pallas import tpu_sc as plsc

NUM_ITEMS = 1000000
D = 32
TROWS = 8
B = 16384
NC = 2
NS = 16
NW = NC * NS
B_PER_W = B // NW
ROUND = 32
N_ROUNDS = B_PER_W // ROUND
BLOCKS = ROUND // 16


def _lane_perm(t, p):
    dnums = lax.GatherDimensionNumbers(
        offset_dims=(), collapsed_slice_dims=(0,), start_index_map=(0,))
    return lax.gather(t, p[:, None], dnums, slice_sizes=(1,),
                      mode=lax.GatherScatterMode.PROMISE_IN_BOUNDS)


def _make_sc_kernel():
    mesh = plsc.VectorSubcoreMesh(core_axis_name="c", subcore_axis_name="s")

    @functools.partial(
        pl.kernel,
        mesh=mesh,
        out_type=jax.ShapeDtypeStruct((B,), jnp.float32),
        scratch_types=[
            pltpu.VMEM((B_PER_W,), jnp.int32),
            pltpu.VMEM((B_PER_W,), jnp.int32),
            pltpu.VMEM((ROUND, TROWS, D), jnp.float32),
            pltpu.VMEM((ROUND, TROWS, D), jnp.float32),
            pltpu.VMEM((D,), jnp.float32),
            pltpu.VMEM((B_PER_W,), jnp.float32),
            pltpu.SemaphoreType.DMA,
            pltpu.SemaphoreType.DMA,
        ],
    )
    def fcf_kernel(st_hbm, sub_hbm, table_hbm, u_hbm, out_hbm,
                   st_v, sub_v, buf_a, buf_b, u_v, out_v, sem_a, sem_b):
        wid = lax.axis_index("s") * NC + lax.axis_index("c")
        base = wid * B_PER_W

        pltpu.sync_copy(st_hbm.at[wid], st_v)
        pltpu.sync_copy(sub_hbm.at[wid], sub_v)
        pltpu.sync_copy(u_hbm, u_v)

        u_lo = u_v[pl.ds(0, 16)]
        u_hi = u_v[pl.ds(16, 16)]
        lane = lax.iota(jnp.int32, 16)
        perms = [lane ^ jnp.int32(s) for s in (1, 2, 4, 8)]

        def fire(r, buf, sem):
            for q in range(ROUND // 16):
                sv = st_v[pl.ds(r * ROUND + q * 16, 16)]
                for j in range(16):
                    start = pl.multiple_of(sv[j], TROWS)
                    pltpu.async_copy(
                        table_hbm.at[pl.ds(start, TROWS)],
                        buf.at[q * 16 + j], sem)

        def drain(buf, sem):
            for j in range(ROUND):
                pltpu.make_async_copy(
                    table_hbm.at[pl.ds(0, TROWS)], buf.at[j], sem).wait()

        def compute(r, buf):
            for g in range(BLOCKS):
                sv = sub_v[pl.ds(r * ROUND + g * 16, 16)]
                acc = jnp.zeros((16,), jnp.float32)
                for i in range(16):
                    k = g * 16 + i
                    s = sv[i]
                    t = (buf[k, s, pl.ds(0, 16)] * u_lo
                         + buf[k, s, pl.ds(16, 16)] * u_hi)
                    for p in perms:
                        t = t + _lane_perm(t, p)
                    acc = jnp.where(lane == i, t, acc)
                out_v[pl.ds(r * ROUND + g * 16, 16)] = (
                    1.0 / (1.0 + jnp.exp(-acc)))

        fire(jnp.int32(0), buf_a, sem_a)

        def super_round(rr, carry):
            r0 = rr * 2
            fire(r0 + 1, buf_b, sem_b)
            drain(buf_a, sem_a)
            compute(r0, buf_a)

            @pl.when(r0 + 2 < N_ROUNDS)
            def _():
                fire(r0 + 2, buf_a, sem_a)

            drain(buf_b, sem_b)
            compute(r0 + 1, buf_b)
            return carry

        lax.fori_loop(0, N_ROUNDS // 2, super_round, jnp.int32(0))

        pltpu.sync_copy(out_v, out_hbm.at[pl.ds(base, B_PER_W)])

    return fcf_kernel


_fcf_sc = _make_sc_kernel()


def kernel(item_indices, item_table, user_embedding):
    idx = item_indices.astype(jnp.int32)
    st = (idx & ~(TROWS - 1)).reshape(NW, B_PER_W)
    sub = (idx & (TROWS - 1)).reshape(NW, B_PER_W)
    u = user_embedding.reshape(D)
    return _fcf_sc(st, sub, item_table, u)


# speedup vs baseline: 1.0603x; 1.0603x over previous
import dataclasses
import functools

import jax
import jax.numpy as jnp
from jax import lax
from jax.experimental import pallas as pl
from jax.experimental.pallas import tpu as pltpu
from jax.experimental.pallas import tpu_sc as plsc

NUM_ITEMS = 1000000
D = 32
B = 16384
LANES = 128
BN = 2048
N_BLOCKS = -(-NUM_ITEMS // BN)
SROWS = N_BLOCKS * (BN // LANES)
NC = 2
NS = 16
NW = NC * NS
B_PER_W = B // NW
CHUNK = 128
N_CHUNKS = B_PER_W // CHUNK


def _scores_tc():
    def body(u_ref, t_ref, o_ref):
        u = u_ref[...]
        for r in range(BN // LANES):
            col = t_ref[:, pl.ds(r * LANES, LANES)]
            s = jnp.dot(u, col, preferred_element_type=jnp.float32)
            o_ref[pl.ds(r, 1), :] = 1.0 / (1.0 + jnp.exp(-s))

    return pl.pallas_call(
        body,
        grid=(N_BLOCKS,),
        in_specs=[
            pl.BlockSpec((1, D), lambda j: (0, 0)),
            pl.BlockSpec((D, BN), lambda j: (0, j)),
        ],
        out_specs=pl.BlockSpec((BN // LANES, LANES), lambda j: (j, 0)),
        out_shape=jax.ShapeDtypeStruct((SROWS, LANES), jnp.float32),
    )


def _gather_sc():
    mesh = plsc.VectorSubcoreMesh(core_axis_name="c", subcore_axis_name="s")
    cp = pltpu.CompilerParams()
    if "needs_layout_passes" in pltpu.CompilerParams.__dataclass_fields__:
        cp = dataclasses.replace(cp, needs_layout_passes=False)

    @functools.partial(
        pl.kernel,
        mesh=mesh,
        compiler_params=cp,
        out_type=jax.ShapeDtypeStruct((B,), jnp.float32),
        scratch_types=[
            pltpu.VMEM((N_CHUNKS, CHUNK), jnp.int32),
            pltpu.VMEM((B_PER_W,), jnp.int32),
            pltpu.VMEM((B_PER_W, LANES), jnp.float32),
            pltpu.VMEM((B_PER_W,), jnp.float32),
            pltpu.SemaphoreType.DMA,
        ],
    )
    def gather_kernel(rw_hbm, ln_hbm, sig_hbm, out_hbm,
                      rw_v, ln_v, rows_v, out_v, sem):
        wid = lax.axis_index("s") * NC + lax.axis_index("c")
        base = wid * B_PER_W

        pltpu.sync_copy(rw_hbm.at[wid], rw_v)
        pltpu.sync_copy(ln_hbm.at[wid], ln_v)

        copies = []
        for j in range(N_CHUNKS):
            copies.append(pltpu.async_copy(
                sig_hbm.at[rw_v.at[j]],
                rows_v.at[pl.ds(j * CHUNK, CHUNK)],
                sem,
            ))
        for c in copies:
            c.wait()

        lane16 = lax.iota(jnp.int32, 16)
        for g in range(B_PER_W // 16):
            kv = g * 16 + lane16
            cv = ln_v[pl.ds(g * 16, 16)]
            out_v[pl.ds(g * 16, 16)] = plsc.load_gather(rows_v, [kv, cv])

        pltpu.sync_copy(out_v, out_hbm.at[pl.ds(base, B_PER_W)])

    return gather_kernel


_tc_scores = _scores_tc()
_sc_gather = _gather_sc()


def kernel(item_indices, item_table, user_embedding):
    idx = item_indices.astype(jnp.int32)
    rw = (idx >> 7).reshape(NW, N_CHUNKS, CHUNK)
    ln = (idx & (LANES - 1)).reshape(NW, B_PER_W)
    sig = _tc_scores(user_embedding.reshape(1, D), item_table.T)
    return _sc_gather(rw, ln, sig)


# speedup vs baseline: 1.8272x; 1.7233x over previous
import dataclasses
import functools

import jax
import jax.numpy as jnp
from jax import lax
from jax.experimental import pallas as pl
from jax.experimental.pallas import tpu as pltpu
from jax.experimental.pallas import tpu_sc as plsc

NUM_ITEMS = 1000000
D = 32
B = 16384
LANES = 128
BN = 4096
N_BLOCKS = -(-NUM_ITEMS // BN)
SROWS = N_BLOCKS * (BN // LANES)
NC = 2
NS = 16
NW = NC * NS
B_PER_W = B // NW
CHUNK = 128
N_CHUNKS = B_PER_W // CHUNK


def _scores_tc():
    def body(u_ref, t_ref, o_ref):
        s = jnp.dot(u_ref[...], t_ref[...],
                    preferred_element_type=jnp.float32)
        sig = 1.0 / (1.0 + jnp.exp(-s))
        for r in range(BN // LANES):
            o_ref[pl.ds(r, 1), :] = sig[:, r * LANES:(r + 1) * LANES]

    return pl.pallas_call(
        body,
        grid=(N_BLOCKS,),
        in_specs=[
            pl.BlockSpec((1, D), lambda j: (0, 0)),
            pl.BlockSpec((D, BN), lambda j: (0, j)),
        ],
        out_specs=pl.BlockSpec((BN // LANES, LANES), lambda j: (j, 0)),
        out_shape=jax.ShapeDtypeStruct((SROWS, LANES), jnp.float32),
    )


def _gather_sc():
    mesh = plsc.VectorSubcoreMesh(core_axis_name="c", subcore_axis_name="s")
    cp = pltpu.CompilerParams()
    if "needs_layout_passes" in pltpu.CompilerParams.__dataclass_fields__:
        cp = dataclasses.replace(cp, needs_layout_passes=False)

    @functools.partial(
        pl.kernel,
        mesh=mesh,
        compiler_params=cp,
        out_type=jax.ShapeDtypeStruct((B,), jnp.float32),
        scratch_types=[
            pltpu.VMEM((N_CHUNKS, CHUNK), jnp.int32),
            pltpu.VMEM((B_PER_W,), jnp.int32),
            pltpu.VMEM((B_PER_W, LANES), jnp.float32),
            pltpu.VMEM((B_PER_W,), jnp.float32),
            pltpu.SemaphoreType.DMA,
        ],
    )
    def gather_kernel(rw_hbm, ln_hbm, sig_hbm, out_hbm,
                      rw_v, ln_v, rows_v, out_v, sem):
        wid = lax.axis_index("s") * NC + lax.axis_index("c")
        base = wid * B_PER_W

        pltpu.sync_copy(rw_hbm.at[wid], rw_v)
        pltpu.sync_copy(ln_hbm.at[wid], ln_v)

        copies = []
        for j in range(N_CHUNKS):
            copies.append(pltpu.async_copy(
                sig_hbm.at[rw_v.at[j]],
                rows_v.at[pl.ds(j * CHUNK, CHUNK)],
                sem,
            ))
        for c in copies:
            c.wait()

        lane16 = lax.iota(jnp.int32, 16)
        for g in range(B_PER_W // 16):
            kv = g * 16 + lane16
            cv = ln_v[pl.ds(g * 16, 16)]
            out_v[pl.ds(g * 16, 16)] = plsc.load_gather(rows_v, [kv, cv])

        pltpu.sync_copy(out_v, out_hbm.at[pl.ds(base, B_PER_W)])

    return gather_kernel


_tc_scores = _scores_tc()
_sc_gather = _gather_sc()


def kernel(item_indices, item_table, user_embedding):
    idx = item_indices.astype(jnp.int32)
    rw = (idx >> 7).reshape(NW, N_CHUNKS, CHUNK)
    ln = (idx & (LANES - 1)).reshape(NW, B_PER_W)
    sig = _tc_scores(user_embedding.reshape(1, D), item_table.T)
    return _sc_gather(rw, ln, sig)


# speedup vs baseline: 5.1501x; 2.8186x over previous
import dataclasses
import functools

import jax
import jax.numpy as jnp
from jax import lax
from jax.experimental import pallas as pl
from jax.experimental.pallas import tpu as pltpu
from jax.experimental.pallas import tpu_sc as plsc

NUM_ITEMS = 1000000
D = 32
B = 16384
LANES = 128
BN = 65536
BSUB = 4096
N_BLOCKS = -(-NUM_ITEMS // BN)
SROWS = N_BLOCKS * (BN // LANES)
NC = 2
NS = 16
NW = NC * NS
B_PER_W = B // NW
CHUNK = 128
N_CHUNKS = B_PER_W // CHUNK


def _scores_tc():
    def body(u_ref, t_ref, o_ref):
        u = u_ref[...]
        for rr in range(BN // BSUB):
            sub = t_ref[:, pl.ds(rr * BSUB, BSUB)]
            s = jnp.dot(u, sub, preferred_element_type=jnp.float32)
            sig = 1.0 / (1.0 + jnp.exp(-s))
            rbase = rr * (BSUB // LANES)
            for r in range(BSUB // LANES):
                o_ref[pl.ds(rbase + r, 1), :] = (
                    sig[:, r * LANES:(r + 1) * LANES])

    return pl.pallas_call(
        body,
        grid=(N_BLOCKS,),
        in_specs=[
            pl.BlockSpec((1, D), lambda j: (0, 0)),
            pl.BlockSpec((D, BN), lambda j: (0, j)),
        ],
        out_specs=pl.BlockSpec((BN // LANES, LANES), lambda j: (j, 0)),
        out_shape=jax.ShapeDtypeStruct((SROWS, LANES), jnp.float32),
    )


def _gather_sc():
    mesh = plsc.VectorSubcoreMesh(core_axis_name="c", subcore_axis_name="s")
    cp = pltpu.CompilerParams()
    if "needs_layout_passes" in pltpu.CompilerParams.__dataclass_fields__:
        cp = dataclasses.replace(cp, needs_layout_passes=False)

    @functools.partial(
        pl.kernel,
        mesh=mesh,
        compiler_params=cp,
        out_type=jax.ShapeDtypeStruct((B,), jnp.float32),
        scratch_types=[
            pltpu.VMEM((N_CHUNKS, CHUNK), jnp.int32),
            pltpu.VMEM((B_PER_W,), jnp.int32),
            pltpu.VMEM((B_PER_W, LANES), jnp.float32),
            pltpu.VMEM((B_PER_W,), jnp.float32),
            pltpu.SemaphoreType.DMA,
        ],
    )
    def gather_kernel(rw_hbm, ln_hbm, sig_hbm, out_hbm,
                      rw_v, ln_v, rows_v, out_v, sem):
        wid = lax.axis_index("s") * NC + lax.axis_index("c")
        base = wid * B_PER_W

        pltpu.sync_copy(rw_hbm.at[wid], rw_v)
        pltpu.sync_copy(ln_hbm.at[wid], ln_v)

        copies = []
        for j in range(N_CHUNKS):
            copies.append(pltpu.async_copy(
                sig_hbm.at[rw_v.at[j]],
                rows_v.at[pl.ds(j * CHUNK, CHUNK)],
                sem,
            ))
        for c in copies:
            c.wait()

        lane16 = lax.iota(jnp.int32, 16)
        for g in range(B_PER_W // 16):
            kv = g * 16 + lane16
            cv = ln_v[pl.ds(g * 16, 16)]
            out_v[pl.ds(g * 16, 16)] = plsc.load_gather(rows_v, [kv, cv])

        pltpu.sync_copy(out_v, out_hbm.at[pl.ds(base, B_PER_W)])

    return gather_kernel


_tc_scores = _scores_tc()
_sc_gather = _gather_sc()


def kernel(item_indices, item_table, user_embedding):
    idx = item_indices.astype(jnp.int32)
    rw = (idx >> 7).reshape(NW, N_CHUNKS, CHUNK)
    ln = (idx & (LANES - 1)).reshape(NW, B_PER_W)
    sig = _tc_scores(user_embedding.reshape(1, D), item_table.T)
    return _sc_gather(rw, ln, sig)
